# Initial kernel scaffold; baseline (speedup 1.0000x reference)
import jax
import jax.numpy as jnp
from jax.experimental import pallas as pl


def kernel(x, edge_index, edge_weight, kernel, bias):
    raise NotImplementedError("write your pallas kernel here")


# speedup vs baseline: 2.1245x; 2.1245x over previous
import dataclasses
import functools

import jax
import jax.numpy as jnp
from jax import lax
from jax.experimental import pallas as pl
from jax.experimental.pallas import tpu as pltpu
from jax.experimental.pallas import tpu_sc as plsc

N_CORES = 2
N_SUBCORES = 16
N_WORKERS = N_CORES * N_SUBCORES
LANES = 16
C_PER_TILE = 4

EDGE_CHUNK = 4000
NBUF = 2


def _scatter_accumulate(xp, edges, n_nodes, n_chunks):
    flat = n_nodes * C_PER_TILE
    mesh = plsc.VectorSubcoreMesh(core_axis_name="c", subcore_axis_name="s")
    cp = pltpu.CompilerParams()
    if "needs_layout_passes" in pltpu.CompilerParams.__dataclass_fields__:
        cp = dataclasses.replace(cp, needs_layout_passes=False)

    @functools.partial(
        pl.kernel,
        compiler_params=cp,
        out_type=jax.ShapeDtypeStruct((N_WORKERS, flat), jnp.float32),
        mesh=mesh,
        scratch_types=[
            pltpu.VMEM((flat,), jnp.float32),
            pltpu.VMEM((flat,), jnp.float32),
            pltpu.VMEM((NBUF, 2, EDGE_CHUNK), jnp.int32),
            pltpu.SemaphoreType.DMA,
            pltpu.SemaphoreType.DMA,
        ],
    )
    def sc_kernel(xp_hbm, e_hbm, o_hbm, x_tile, acc, ebuf, sem0, sem1):
        sems = (sem0, sem1)
        wid = lax.axis_index("c") * N_SUBCORES + lax.axis_index("s")

        pltpu.sync_copy(xp_hbm.at[wid], x_tile)

        zeros = jnp.zeros((LANES,), jnp.float32)

        @pl.loop(0, flat, step=LANES)
        def _(i):
            acc[pl.ds(i, LANES)] = zeros

        for b in range(NBUF):
            pltpu.make_async_copy(e_hbm.at[b], ebuf.at[b], sems[b]).start()

        def process(buf):
            @pl.loop(0, EDGE_CHUNK, step=LANES)
            def _(i):
                pk = buf[0, pl.ds(i, LANES)]
                wv = plsc.bitcast(buf[1, pl.ds(i, LANES)], jnp.float32)
                s4 = (pk >> 14) << 2
                d4 = (pk & 16383) << 2
                for j in range(C_PER_TILE):
                    sj = s4 if j == 0 else s4 + j
                    dj = d4 if j == 0 else d4 + j
                    v = plsc.load_gather(x_tile, [sj])
                    plsc.addupdate_scatter(acc, [dj], v * wv)

        @pl.loop(0, n_chunks, step=NBUF)
        def _(c):
            for b in range(NBUF):
                cur = c + b
                pltpu.make_async_copy(e_hbm.at[cur], ebuf.at[b], sems[b]).wait()
                process(ebuf.at[b])
                nxt = cur + NBUF

                @pl.when(nxt < n_chunks)
                def _():
                    pltpu.make_async_copy(
                        e_hbm.at[nxt], ebuf.at[b], sems[b]
                    ).start()

        pltpu.sync_copy(acc, o_hbm.at[wid])

    return sc_kernel(xp, edges)


def _project(s, w, b, blk):
    n, d = s.shape
    c = w.shape[1]

    def body(s_ref, w_ref, b_ref, o_ref):
        o_ref[...] = (
            jnp.dot(s_ref[...], w_ref[...], preferred_element_type=jnp.float32)
            + b_ref[...]
        )

    return pl.pallas_call(
        body,
        grid=(n // blk,),
        in_specs=[
            pl.BlockSpec((blk, d), lambda i: (i, 0)),
            pl.BlockSpec((d, c), lambda i: (0, 0)),
            pl.BlockSpec((1, c), lambda i: (0, 0)),
        ],
        out_specs=pl.BlockSpec((blk, c), lambda i: (i, 0)),
        out_shape=jax.ShapeDtypeStruct((n, c), jnp.float32),
    )(s, w, b.reshape(1, c))


def kernel(x, edge_index, edge_weight, kernel, bias):
    n_nodes, d_feat = x.shape
    channels = kernel.shape[1]
    n_edges = edge_index.shape[1]

    dst = edge_index[0].astype(jnp.int32)
    src = edge_index[1].astype(jnp.int32)
    pack = (src << 14) | dst
    wbits = lax.bitcast_convert_type(edge_weight.astype(jnp.float32), jnp.int32)

    n_chunks = -(-n_edges // EDGE_CHUNK)
    n_chunks += n_chunks % NBUF
    e_pad = n_chunks * EDGE_CHUNK
    pad = e_pad - n_edges
    if pad:
        pack = jnp.pad(pack, (0, pad))
        wbits = jnp.pad(wbits, (0, pad))
    edges = jnp.stack(
        [pack.reshape(n_chunks, EDGE_CHUNK), wbits.reshape(n_chunks, EDGE_CHUNK)],
        axis=1,
    )

    xp = (
        x.reshape(n_nodes, N_WORKERS, C_PER_TILE)
        .transpose(1, 0, 2)
        .reshape(N_WORKERS, n_nodes * C_PER_TILE)
    )

    sp = _scatter_accumulate(xp, edges, n_nodes, n_chunks)
    s = (
        sp.reshape(N_WORKERS, n_nodes, C_PER_TILE)
        .transpose(1, 0, 2)
        .reshape(n_nodes, d_feat)
    )

    return _project(s, kernel, bias, blk=2000)


# speedup vs baseline: 4.4080x; 2.0749x over previous
import dataclasses
import functools

import jax
import jax.numpy as jnp
from jax import lax
from jax.experimental import pallas as pl
from jax.experimental.pallas import tpu as pltpu
from jax.experimental.pallas import tpu_sc as plsc

N_CORES = 2
N_SUBCORES = 16
N_WORKERS = N_CORES * N_SUBCORES
LANES = 16
C_PER_TILE = 4

EDGE_CHUNK = 4096
NBUF = 2
GROUP_UNROLL = 4


def _scatter_accumulate(xp, edges, n_nodes, n_chunks):
    flat = n_nodes * C_PER_TILE
    mesh = plsc.VectorSubcoreMesh(core_axis_name="c", subcore_axis_name="s")
    cp = pltpu.CompilerParams()
    if "needs_layout_passes" in pltpu.CompilerParams.__dataclass_fields__:
        cp = dataclasses.replace(cp, needs_layout_passes=False)

    @functools.partial(
        pl.kernel,
        compiler_params=cp,
        out_type=jax.ShapeDtypeStruct((N_WORKERS, flat), jnp.float32),
        mesh=mesh,
        scratch_types=[
            pltpu.VMEM((flat,), jnp.float32),
            pltpu.VMEM((flat,), jnp.float32),
            pltpu.VMEM((NBUF, 2, EDGE_CHUNK), jnp.int32),
            pltpu.SemaphoreType.DMA,
            pltpu.SemaphoreType.DMA,
        ],
    )
    def sc_kernel(xp_hbm, e_hbm, o_hbm, x_tile, acc, ebuf, sem0, sem1):
        sems = (sem0, sem1)
        wid = lax.axis_index("c") * N_SUBCORES + lax.axis_index("s")

        pltpu.sync_copy(xp_hbm.at[wid], x_tile)

        zeros = jnp.zeros((LANES,), jnp.float32)

        @pl.loop(0, flat, step=LANES)
        def _(i):
            acc[pl.ds(i, LANES)] = zeros

        for b in range(NBUF):
            pltpu.make_async_copy(e_hbm.at[b], ebuf.at[b], sems[b]).start()

        def process(buf):
            @plsc.parallel_loop(0, EDGE_CHUNK, step=LANES, unroll=GROUP_UNROLL)
            def _(i):
                pk = buf[0, pl.ds(i, LANES)]
                wv = plsc.bitcast(buf[1, pl.ds(i, LANES)], jnp.float32)
                s4 = (pk >> 14) << 2
                d4 = (pk & 16383) << 2
                for j in range(C_PER_TILE):
                    sj = s4 if j == 0 else s4 + j
                    dj = d4 if j == 0 else d4 + j
                    v = plsc.load_gather(x_tile, [sj])
                    plsc.addupdate_scatter(acc, [dj], v * wv)

        @pl.loop(0, n_chunks, step=NBUF)
        def _(c):
            for b in range(NBUF):
                cur = c + b
                pltpu.make_async_copy(e_hbm.at[cur], ebuf.at[b], sems[b]).wait()
                process(ebuf.at[b])
                nxt = cur + NBUF

                @pl.when(nxt < n_chunks)
                def _():
                    pltpu.make_async_copy(
                        e_hbm.at[nxt], ebuf.at[b], sems[b]
                    ).start()

        pltpu.sync_copy(acc, o_hbm.at[wid])

    return sc_kernel(xp, edges)


def _project(s, w, b, blk):
    n, d = s.shape
    c = w.shape[1]

    def body(s_ref, w_ref, b_ref, o_ref):
        o_ref[...] = (
            jnp.dot(s_ref[...], w_ref[...], preferred_element_type=jnp.float32)
            + b_ref[...]
        )

    return pl.pallas_call(
        body,
        grid=(n // blk,),
        in_specs=[
            pl.BlockSpec((blk, d), lambda i: (i, 0)),
            pl.BlockSpec((d, c), lambda i: (0, 0)),
            pl.BlockSpec((1, c), lambda i: (0, 0)),
        ],
        out_specs=pl.BlockSpec((blk, c), lambda i: (i, 0)),
        out_shape=jax.ShapeDtypeStruct((n, c), jnp.float32),
    )(s, w, b.reshape(1, c))


def kernel(x, edge_index, edge_weight, kernel, bias):
    n_nodes, d_feat = x.shape
    channels = kernel.shape[1]
    n_edges = edge_index.shape[1]

    dst = edge_index[0].astype(jnp.int32)
    src = edge_index[1].astype(jnp.int32)
    pack = (src << 14) | dst
    wbits = lax.bitcast_convert_type(edge_weight.astype(jnp.float32), jnp.int32)

    n_chunks = -(-n_edges // EDGE_CHUNK)
    n_chunks += n_chunks % NBUF
    e_pad = n_chunks * EDGE_CHUNK
    pad = e_pad - n_edges
    if pad:
        pack = jnp.pad(pack, (0, pad))
        wbits = jnp.pad(wbits, (0, pad))
    edges = jnp.stack(
        [pack.reshape(n_chunks, EDGE_CHUNK), wbits.reshape(n_chunks, EDGE_CHUNK)],
        axis=1,
    )

    xp = (
        x.reshape(n_nodes, N_WORKERS, C_PER_TILE)
        .transpose(1, 0, 2)
        .reshape(N_WORKERS, n_nodes * C_PER_TILE)
    )

    sp = _scatter_accumulate(xp, edges, n_nodes, n_chunks)
    s = (
        sp.reshape(N_WORKERS, n_nodes, C_PER_TILE)
        .transpose(1, 0, 2)
        .reshape(n_nodes, d_feat)
    )

    return _project(s, kernel, bias, blk=2000)


# speedup vs baseline: 6.7940x; 1.5413x over previous
import dataclasses
import functools

import jax
import jax.numpy as jnp
from jax import lax
from jax.experimental import pallas as pl
from jax.experimental.pallas import tpu as pltpu
from jax.experimental.pallas import tpu_sc as plsc

N_CORES = 2
N_SUBCORES = 16
N_WORKERS = N_CORES * N_SUBCORES
LANES = 16
C_PER_TILE = 4

EDGE_CHUNK = 8192
NBUF = 2
GROUP_UNROLL = 4


def _scatter_accumulate(xt, edges, n_nodes, n_chunks):
    flat = n_nodes * C_PER_TILE
    mesh = plsc.VectorSubcoreMesh(core_axis_name="c", subcore_axis_name="s")
    cp = pltpu.CompilerParams()
    if "needs_layout_passes" in pltpu.CompilerParams.__dataclass_fields__:
        cp = dataclasses.replace(cp, needs_layout_passes=False)

    @functools.partial(
        pl.kernel,
        compiler_params=cp,
        out_type=jax.ShapeDtypeStruct((N_WORKERS, flat), jnp.float32),
        mesh=mesh,
        scratch_types=[
            pltpu.VMEM((flat,), jnp.float32),
            pltpu.VMEM((flat,), jnp.float32),
            pltpu.VMEM((NBUF, 2, EDGE_CHUNK), jnp.int32),
            pltpu.SemaphoreType.DMA,
            pltpu.SemaphoreType.DMA,
        ],
    )
    def sc_kernel(xt_hbm, e_hbm, o_hbm, x_tile, acc, ebuf, sem0, sem1):
        sems = (sem0, sem1)
        wid = lax.axis_index("c") * N_SUBCORES + lax.axis_index("s")

        pltpu.sync_copy(xt_hbm.at[wid], x_tile)

        zeros = jnp.zeros((LANES,), jnp.float32)

        @pl.loop(0, flat, step=LANES)
        def _(i):
            acc[pl.ds(i, LANES)] = zeros

        for b in range(NBUF):
            pltpu.make_async_copy(e_hbm.at[b], ebuf.at[b], sems[b]).start()

        def process(buf):
            @plsc.parallel_loop(0, EDGE_CHUNK, step=LANES, unroll=GROUP_UNROLL)
            def _(i):
                pk = buf[0, pl.ds(i, LANES)]
                wv = plsc.bitcast(buf[1, pl.ds(i, LANES)], jnp.float32)
                s = pk >> 14
                d = pk & 16383
                for j in range(C_PER_TILE):
                    sj = s if j == 0 else s + (j * n_nodes)
                    dj = d if j == 0 else d + (j * n_nodes)
                    v = plsc.load_gather(x_tile, [sj])
                    plsc.addupdate_scatter(acc, [dj], v * wv)

        @pl.loop(0, n_chunks, step=NBUF)
        def _(c):
            for b in range(NBUF):
                cur = c + b
                pltpu.make_async_copy(e_hbm.at[cur], ebuf.at[b], sems[b]).wait()
                process(ebuf.at[b])
                nxt = cur + NBUF

                @pl.when(nxt < n_chunks)
                def _():
                    pltpu.make_async_copy(
                        e_hbm.at[nxt], ebuf.at[b], sems[b]
                    ).start()

        pltpu.sync_copy(acc, o_hbm.at[wid])

    return sc_kernel(xt, edges)


def _project(st, w, b):
    d, n = st.shape
    c = w.shape[1]

    def body(st_ref, w_ref, b_ref, o_ref):
        o_ref[...] = (
            lax.dot_general(
                st_ref[...],
                w_ref[...],
                dimension_numbers=(((0,), (0,)), ((), ())),
                preferred_element_type=jnp.float32,
            )
            + b_ref[...]
        )

    return pl.pallas_call(
        body,
        out_shape=jax.ShapeDtypeStruct((n, c), jnp.float32),
    )(st, w, b.reshape(1, c))


def kernel(x, edge_index, edge_weight, kernel, bias):
    n_nodes, d_feat = x.shape
    n_edges = edge_index.shape[1]

    dst = edge_index[0].astype(jnp.int32)
    src = edge_index[1].astype(jnp.int32)
    pack = (src << 14) | dst
    wbits = lax.bitcast_convert_type(edge_weight.astype(jnp.float32), jnp.int32)

    n_chunks = -(-n_edges // EDGE_CHUNK)
    n_chunks += n_chunks % NBUF
    e_pad = n_chunks * EDGE_CHUNK
    pad = e_pad - n_edges
    if pad:
        pack = jnp.pad(pack, (0, pad))
        wbits = jnp.pad(wbits, (0, pad))
    edges = jnp.stack(
        [pack.reshape(n_chunks, EDGE_CHUNK), wbits.reshape(n_chunks, EDGE_CHUNK)],
        axis=1,
    )

    xt = x.T.reshape(N_WORKERS, C_PER_TILE * n_nodes)

    st = _scatter_accumulate(xt, edges, n_nodes, n_chunks).reshape(d_feat, n_nodes)

    return _project(st, kernel, bias)


# speedup vs baseline: 8.0754x; 1.1886x over previous
import dataclasses
import functools

import jax
import jax.numpy as jnp
from jax import lax
from jax.experimental import pallas as pl
from jax.experimental.pallas import tpu as pltpu
from jax.experimental.pallas import tpu_sc as plsc

N_CORES = 2
N_SUBCORES = 16
N_WORKERS = N_CORES * N_SUBCORES
LANES = 16
C_PER_TILE = 4

EDGE_CHUNK = 8000
NBUF = 2
GROUP_UNROLL = 4


def _scatter_accumulate(xt, edges, n_nodes, n_chunks):
    mesh = plsc.VectorSubcoreMesh(core_axis_name="c", subcore_axis_name="s")
    cp = pltpu.CompilerParams()
    if "needs_layout_passes" in pltpu.CompilerParams.__dataclass_fields__:
        cp = dataclasses.replace(cp, needs_layout_passes=False)

    @functools.partial(
        pl.kernel,
        compiler_params=cp,
        out_type=jax.ShapeDtypeStruct((N_WORKERS * C_PER_TILE, n_nodes), jnp.float32),
        mesh=mesh,
        scratch_types=(
            [pltpu.VMEM((n_nodes,), jnp.float32) for _ in range(2 * C_PER_TILE)]
            + [
                pltpu.VMEM((NBUF, 2, EDGE_CHUNK), jnp.int32),
                pltpu.SemaphoreType.DMA,
                pltpu.SemaphoreType.DMA,
            ]
        ),
    )
    def sc_kernel(xt_hbm, e_hbm, o_hbm, x0, x1, x2, x3, a0, a1, a2, a3,
                  ebuf, sem0, sem1):
        xs = (x0, x1, x2, x3)
        accs = (a0, a1, a2, a3)
        sems = (sem0, sem1)
        wid = lax.axis_index("c") * N_SUBCORES + lax.axis_index("s")
        row0 = wid * C_PER_TILE

        for j in range(C_PER_TILE):
            pltpu.sync_copy(xt_hbm.at[row0 + j], xs[j])

        zeros = jnp.zeros((LANES,), jnp.float32)

        @pl.loop(0, n_nodes, step=LANES)
        def _(i):
            for j in range(C_PER_TILE):
                accs[j][pl.ds(i, LANES)] = zeros

        for b in range(NBUF):
            pltpu.make_async_copy(e_hbm.at[b], ebuf.at[b], sems[b]).start()

        def process(buf):
            @plsc.parallel_loop(0, EDGE_CHUNK, step=LANES, unroll=GROUP_UNROLL)
            def _(i):
                pk = buf[0, pl.ds(i, LANES)]
                wv = plsc.bitcast(buf[1, pl.ds(i, LANES)], jnp.float32)
                s = pk >> 14
                d = pk & 16383
                for j in range(C_PER_TILE):
                    v = plsc.load_gather(xs[j], [s])
                    plsc.addupdate_scatter(accs[j], [d], v * wv)

        @pl.loop(0, n_chunks, step=NBUF)
        def _(c):
            for b in range(NBUF):
                cur = c + b
                pltpu.make_async_copy(e_hbm.at[cur], ebuf.at[b], sems[b]).wait()
                process(ebuf.at[b])
                nxt = cur + NBUF

                @pl.when(nxt < n_chunks)
                def _():
                    pltpu.make_async_copy(
                        e_hbm.at[nxt], ebuf.at[b], sems[b]
                    ).start()

        for j in range(C_PER_TILE):
            pltpu.sync_copy(accs[j], o_hbm.at[row0 + j])

    return sc_kernel(xt, edges)


def _project(st, w, b):
    d, n = st.shape
    c = w.shape[1]

    def body(st_ref, w_ref, b_ref, o_ref):
        o_ref[...] = (
            lax.dot_general(
                st_ref[...],
                w_ref[...],
                dimension_numbers=(((0,), (0,)), ((), ())),
                preferred_element_type=jnp.float32,
            )
            + b_ref[...]
        )

    return pl.pallas_call(
        body,
        out_shape=jax.ShapeDtypeStruct((n, c), jnp.float32),
    )(st, w, b.reshape(1, c))


def kernel(x, edge_index, edge_weight, kernel, bias):
    n_nodes, d_feat = x.shape
    n_edges = edge_index.shape[1]

    dst = edge_index[0].astype(jnp.int32)
    src = edge_index[1].astype(jnp.int32)
    pack = (src << 14) | dst
    wbits = lax.bitcast_convert_type(edge_weight.astype(jnp.float32), jnp.int32)

    n_chunks = -(-n_edges // EDGE_CHUNK)
    n_chunks += n_chunks % NBUF
    e_pad = n_chunks * EDGE_CHUNK
    pad = e_pad - n_edges
    if pad:
        pack = jnp.pad(pack, (0, pad))
        wbits = jnp.pad(wbits, (0, pad))
    edges = jnp.stack(
        [pack.reshape(n_chunks, EDGE_CHUNK), wbits.reshape(n_chunks, EDGE_CHUNK)],
        axis=1,
    )

    xt = x.T

    st = _scatter_accumulate(xt, edges, n_nodes, n_chunks)

    return _project(st, kernel, bias)


# speedup vs baseline: 8.1545x; 1.0098x over previous
import dataclasses
import functools

import jax
import jax.numpy as jnp
from jax import lax
from jax.experimental import pallas as pl
from jax.experimental.pallas import tpu as pltpu
from jax.experimental.pallas import tpu_sc as plsc

N_CORES = 2
N_SUBCORES = 16
N_WORKERS = N_CORES * N_SUBCORES
LANES = 16
C_PER_TILE = 4

EDGE_CHUNK = 8000
NBUF = 2
GROUP_UNROLL = 4


def _scatter_accumulate(xt, edges, n_nodes, n_chunks):
    mesh = plsc.VectorSubcoreMesh(core_axis_name="c", subcore_axis_name="s")
    cp = pltpu.CompilerParams()
    if "needs_layout_passes" in pltpu.CompilerParams.__dataclass_fields__:
        cp = dataclasses.replace(cp, needs_layout_passes=False)

    n_pairs = C_PER_TILE // 2

    @functools.partial(
        pl.kernel,
        compiler_params=cp,
        out_type=jax.ShapeDtypeStruct((N_WORKERS * C_PER_TILE, n_nodes), jnp.float32),
        mesh=mesh,
        scratch_types=(
            [pltpu.VMEM((n_nodes,), jnp.int32) for _ in range(n_pairs)]
            + [pltpu.VMEM((n_nodes,), jnp.float32) for _ in range(C_PER_TILE)]
            + [
                pltpu.VMEM((NBUF, 2, EDGE_CHUNK), jnp.int32),
                pltpu.SemaphoreType.DMA,
                pltpu.SemaphoreType.DMA,
            ]
        ),
    )
    def sc_kernel(xt_hbm, e_hbm, o_hbm, xp0, xp1, a0, a1, a2, a3,
                  ebuf, sem0, sem1):
        xps = (xp0, xp1)
        accs = (a0, a1, a2, a3)
        sems = (sem0, sem1)
        wid = lax.axis_index("c") * N_SUBCORES + lax.axis_index("s")
        pair0 = wid * n_pairs

        for j in range(n_pairs):
            pltpu.sync_copy(xt_hbm.at[pair0 + j], xps[j])

        zeros = jnp.zeros((LANES,), jnp.float32)

        @pl.loop(0, n_nodes, step=LANES)
        def _(i):
            for j in range(C_PER_TILE):
                accs[j][pl.ds(i, LANES)] = zeros

        for b in range(NBUF):
            pltpu.make_async_copy(e_hbm.at[b], ebuf.at[b], sems[b]).start()

        def process(buf):
            @plsc.parallel_loop(0, EDGE_CHUNK, step=LANES, unroll=GROUP_UNROLL)
            def _(i):
                pk = buf[0, pl.ds(i, LANES)]
                wv = plsc.bitcast(buf[1, pl.ds(i, LANES)], jnp.float32)
                s = pk >> 14
                d = pk & 16383
                himask = jnp.int32(-65536)
                for j in range(n_pairs):
                    g = plsc.load_gather(xps[j], [s])
                    vlo = plsc.bitcast(g << 16, jnp.float32)
                    vhi = plsc.bitcast(g & himask, jnp.float32)
                    plsc.addupdate_scatter(accs[2 * j], [d], vlo * wv)
                    plsc.addupdate_scatter(accs[2 * j + 1], [d], vhi * wv)

        @pl.loop(0, n_chunks, step=NBUF)
        def _(c):
            for b in range(NBUF):
                cur = c + b
                pltpu.make_async_copy(e_hbm.at[cur], ebuf.at[b], sems[b]).wait()
                process(ebuf.at[b])
                nxt = cur + NBUF

                @pl.when(nxt < n_chunks)
                def _():
                    pltpu.make_async_copy(
                        e_hbm.at[nxt], ebuf.at[b], sems[b]
                    ).start()

        for j in range(C_PER_TILE):
            pltpu.sync_copy(accs[j], o_hbm.at[wid * C_PER_TILE + j])

    return sc_kernel(xt, edges)


def _project(st, w, b):
    d, n = st.shape
    c = w.shape[1]

    def body(st_ref, w_ref, b_ref, o_ref):
        o_ref[...] = (
            lax.dot_general(
                st_ref[...],
                w_ref[...],
                dimension_numbers=(((0,), (0,)), ((), ())),
                preferred_element_type=jnp.float32,
            )
            + b_ref[...]
        )

    return pl.pallas_call(
        body,
        out_shape=jax.ShapeDtypeStruct((n, c), jnp.float32),
    )(st, w, b.reshape(1, c))


def kernel(x, edge_index, edge_weight, kernel, bias):
    n_nodes, d_feat = x.shape
    n_edges = edge_index.shape[1]

    dst = edge_index[0].astype(jnp.int32)
    src = edge_index[1].astype(jnp.int32)
    pack = (src << 14) | dst
    wbits = lax.bitcast_convert_type(edge_weight.astype(jnp.float32), jnp.int32)

    n_chunks = -(-n_edges // EDGE_CHUNK)
    n_chunks += n_chunks % NBUF
    e_pad = n_chunks * EDGE_CHUNK
    pad = e_pad - n_edges
    if pad:
        pack = jnp.pad(pack, (0, pad))
        wbits = jnp.pad(wbits, (0, pad))
    edges = jnp.stack(
        [pack.reshape(n_chunks, EDGE_CHUNK), wbits.reshape(n_chunks, EDGE_CHUNK)],
        axis=1,
    )

    xb = x.T.astype(jnp.bfloat16)
    xu = lax.bitcast_convert_type(xb, jnp.uint16).astype(jnp.uint32)
    xt = lax.bitcast_convert_type(
        (xu[1::2] << 16) | xu[0::2], jnp.int32
    )

    st = _scatter_accumulate(xt, edges, n_nodes, n_chunks)

    return _project(st, kernel, bias)
